# Initial kernel scaffold; baseline (speedup 1.0000x reference)
import jax
import jax.numpy as jnp
from jax.experimental import pallas as pl


def kernel(x, edge_index, avg_deg_log, W_pre, W_lin, bias):
    raise NotImplementedError("write your pallas kernel here")


# speedup vs baseline: 6.0394x; 6.0394x over previous
import functools

import jax
import jax.numpy as jnp
from jax import lax
from jax.experimental import pallas as pl
from jax.experimental.pallas import tpu as pltpu
from jax.experimental.pallas import tpu_sc as plsc

N = 10000
E = 320000
D = 128
OUT = 128
EPS = 1e-07

NC = 2
NS = 16
CHUNK = 80
NROWS = E // CHUNK
ROWS_TILE = 256
ROWS_LAST = NROWS - ROWS_TILE * (NS - 1)
NSPLIT = 640
NLAST = N - NSPLIT * (NS - 1)
STAGE = 32


def _square_stack_body(x_ref, out_ref):
    xb = x_ref[...]
    out_ref[0] = xb
    out_ref[1] = xb * xb


def _square_stack(x):
    blk = 2000
    return pl.pallas_call(
        _square_stack_body,
        grid=(N // blk,),
        in_specs=[pl.BlockSpec((blk, D), lambda i: (i, 0))],
        out_specs=pl.BlockSpec((2, blk, D), lambda i: (0, i, 0)),
        out_shape=jax.ShapeDtypeStruct((2, N, D), jnp.float32),
    )(x)


def _seg_body(xs_hbm, row_hbm, col_hbm, s1_hbm, s2_hbm, deg_hbm,
              row_v, col_v, gbuf, zb, ones_v, acc, dacc, sem):
    cid = lax.axis_index("c")
    sid = lax.axis_index("s")
    last = sid == NS - 1
    off = cid * N

    zero16 = jnp.zeros((16,), jnp.float32)
    one16 = jnp.full((16,), 1.0, jnp.float32)

    def _zg(t, _):
        r = t // (D // 16)
        k = (t % (D // 16)) * 16
        gbuf[r, pl.ds(k, 16)] = zero16
        return 0
    lax.fori_loop(0, CHUNK * (D // 16), _zg, 0)

    def _z1(t, _):
        zb[t, pl.ds(0, 16)] = zero16
        ones_v[t, pl.ds(0, 16)] = one16
        return 0
    lax.fori_loop(0, CHUNK, _z1, 0)

    nbase = sid * NSPLIT
    nz = jnp.where(last, NLAST // CHUNK, NSPLIT // CHUNK)

    def _zacc(t, _):
        pltpu.sync_copy(gbuf, acc.at[pl.ds(nbase + t * CHUNK, CHUNK)])
        pltpu.sync_copy(zb, dacc.at[pl.ds(nbase + t * CHUNK, CHUNK)])
        return 0
    lax.fori_loop(0, nz, _zacc, 0)

    plsc.subcore_barrier()

    nstages = jnp.where(last, ROWS_LAST // STAGE, ROWS_TILE // STAGE)

    def _stage(st, _):
        rb = sid * ROWS_TILE + st * STAGE
        pltpu.sync_copy(row_hbm.at[pl.ds(rb, STAGE)], row_v)
        pltpu.sync_copy(col_hbm.at[pl.ds(rb, STAGE)], col_v)

        def _offs(t, _):
            r = t // (CHUNK // 16)
            k = (t % (CHUNK // 16)) * 16
            col_v[r, pl.ds(k, 16)] = col_v[r, pl.ds(k, 16)] + off
            return 0
        lax.fori_loop(0, STAGE * (CHUNK // 16), _offs, 0)

        def _edge(j, _):
            pltpu.async_copy(xs_hbm.at[col_v.at[j]], gbuf, sem).wait()
            pltpu.sync_copy(gbuf, acc.at[row_v.at[j]], add=True)

            @pl.when(cid == 0)
            def _():
                pltpu.sync_copy(ones_v, dacc.at[row_v.at[j]], add=True)
            return 0
        lax.fori_loop(0, STAGE, _edge, 0)
        return 0
    lax.fori_loop(0, nstages, _stage, 0)

    plsc.subcore_barrier()

    @pl.when((cid == 0) & jnp.logical_not(last))
    def _():
        pltpu.sync_copy(acc.at[pl.ds(nbase, NSPLIT)], s1_hbm.at[pl.ds(nbase, NSPLIT)])
        pltpu.sync_copy(dacc.at[pl.ds(nbase, NSPLIT)], deg_hbm.at[pl.ds(nbase, NSPLIT)])

    @pl.when((cid == 0) & last)
    def _():
        pltpu.sync_copy(acc.at[pl.ds(nbase, NLAST)], s1_hbm.at[pl.ds(nbase, NLAST)])
        pltpu.sync_copy(dacc.at[pl.ds(nbase, NLAST)], deg_hbm.at[pl.ds(nbase, NLAST)])

    @pl.when((cid == 1) & jnp.logical_not(last))
    def _():
        pltpu.sync_copy(acc.at[pl.ds(nbase, NSPLIT)], s2_hbm.at[pl.ds(nbase, NSPLIT)])

    @pl.when((cid == 1) & last)
    def _():
        pltpu.sync_copy(acc.at[pl.ds(nbase, NLAST)], s2_hbm.at[pl.ds(nbase, NLAST)])


_seg_kernel = functools.partial(
    pl.kernel,
    out_type=(jax.ShapeDtypeStruct((N, D), jnp.float32),
              jax.ShapeDtypeStruct((N, D), jnp.float32),
              jax.ShapeDtypeStruct((N, 16), jnp.float32)),
    mesh=plsc.VectorSubcoreMesh(core_axis_name="c", subcore_axis_name="s",
                                num_cores=NC, num_subcores=NS),
    compiler_params=pltpu.CompilerParams(use_tc_tiling_on_sc=False),
    scratch_types=[
        pltpu.VMEM((STAGE, CHUNK), jnp.int32),
        pltpu.VMEM((STAGE, CHUNK), jnp.int32),
        pltpu.VMEM((CHUNK, D), jnp.float32),
        pltpu.VMEM((CHUNK, 16), jnp.float32),
        pltpu.VMEM((CHUNK, 16), jnp.float32),
        pltpu.VMEM_SHARED((N, D), jnp.float32),
        pltpu.VMEM_SHARED((N, 16), jnp.float32),
        pltpu.SemaphoreType.DMA,
    ],
)(_seg_body)


def _epi_body(avg_ref, s1_ref, s2_ref, deg_ref, x_ref,
              wid_ref, wamp_ref, watt_ref, bias_ref, out_ref):
    avg = avg_ref[0, 0]
    s1 = s1_ref[...]
    s2 = s2_ref[...]
    x = x_ref[...]
    std = jnp.sqrt(jnp.maximum(s2 - s1 * s1, 0.0) + EPS)
    logd = jnp.log(deg_ref[:, 0:1] + 1.0)
    amp = logd / avg
    att = avg / (logd + EPS)

    f32 = jnp.float32
    h_id = (jnp.dot(s1, wid_ref[0:D], preferred_element_type=f32)
            + jnp.dot(std, wid_ref[D:2 * D], preferred_element_type=f32)
            + jnp.dot(x, wid_ref[2 * D:3 * D], preferred_element_type=f32))
    h_amp = (jnp.dot(s1, wamp_ref[0:D], preferred_element_type=f32)
             + jnp.dot(std, wamp_ref[D:2 * D], preferred_element_type=f32))
    h_att = (jnp.dot(s1, watt_ref[0:D], preferred_element_type=f32)
             + jnp.dot(std, watt_ref[D:2 * D], preferred_element_type=f32))
    out_ref[...] = h_id + amp * h_amp + att * h_att + bias_ref[...]


def _epilogue(avg, s1, s2, deg, x, w_id, w_amp, w_att, bias):
    blk = 2000
    return pl.pallas_call(
        _epi_body,
        grid=(N // blk,),
        in_specs=[
            pl.BlockSpec(memory_space=pltpu.SMEM),
            pl.BlockSpec((blk, D), lambda i: (i, 0)),
            pl.BlockSpec((blk, D), lambda i: (i, 0)),
            pl.BlockSpec((blk, 16), lambda i: (i, 0)),
            pl.BlockSpec((blk, D), lambda i: (i, 0)),
            pl.BlockSpec((3 * D, OUT), lambda i: (0, 0)),
            pl.BlockSpec((2 * D, OUT), lambda i: (0, 0)),
            pl.BlockSpec((2 * D, OUT), lambda i: (0, 0)),
            pl.BlockSpec((1, OUT), lambda i: (0, 0)),
        ],
        out_specs=pl.BlockSpec((blk, OUT), lambda i: (i, 0)),
        out_shape=jax.ShapeDtypeStruct((N, OUT), jnp.float32),
    )(avg, s1, s2, deg, x, w_id, w_amp, w_att, bias)


def kernel(x, edge_index, avg_deg_log, W_pre, W_lin, bias):
    row2 = edge_index[0].reshape(NROWS, CHUNK)
    col2 = edge_index[1].reshape(NROWS, CHUNK)

    xs = _square_stack(x).reshape(2 * N, D)
    s1, s2, deg = _seg_kernel(xs, row2, col2)

    w_id = jnp.concatenate([W_pre[0].T, W_pre[3].T, W_lin.T], axis=0)
    w_amp = jnp.concatenate([W_pre[1].T, W_pre[4].T], axis=0)
    w_att = jnp.concatenate([W_pre[2].T, W_pre[5].T], axis=0)
    avg = jnp.reshape(avg_deg_log, (1, 1))
    return _epilogue(avg, s1, s2, deg, x,
                     w_id, w_amp, w_att, bias.reshape(1, OUT))


# speedup vs baseline: 10.2086x; 1.6903x over previous
import functools

import jax
import jax.numpy as jnp
from jax import lax
from jax.experimental import pallas as pl
from jax.experimental.pallas import tpu as pltpu
from jax.experimental.pallas import tpu_sc as plsc

N = 10000
E = 320000
D = 128
OUT = 128
EPS = 1e-07

NC = 2
NS = 16
CHUNK = 64
NROWS = E // CHUNK
ROWS_TILE = 320
ROWS_LAST = NROWS - ROWS_TILE * (NS - 1)
NSPLIT = 640
NLAST = N - NSPLIT * (NS - 1)
STAGE = 40
RING = 4


def _square_stack_body(x_ref, out_ref):
    xb = x_ref[...]
    out_ref[0] = xb
    out_ref[1] = xb * xb


def _square_stack(x):
    blk = 2000
    return pl.pallas_call(
        _square_stack_body,
        grid=(N // blk,),
        in_specs=[pl.BlockSpec((blk, D), lambda i: (i, 0))],
        out_specs=pl.BlockSpec((2, blk, D), lambda i: (0, i, 0)),
        out_shape=jax.ShapeDtypeStruct((2, N, D), jnp.float32),
    )(x)


def _seg_body(xs_hbm, row_hbm, col_hbm, colN_hbm, s1_hbm, s2_hbm, deg_hbm,
              row_v, col_v, gb0, gb1, gb2, gb3, zb, ones_v, acc, dacc,
              gsems, ssems, dsem):
    cid = lax.axis_index("c")
    sid = lax.axis_index("s")
    last = sid == NS - 1
    gbufs = (gb0, gb1, gb2, gb3)

    zero16 = jnp.zeros((16,), jnp.float32)
    one16 = jnp.full((16,), 1.0, jnp.float32)

    def _zg(t, _):
        r = t // (D // 16)
        k = (t % (D // 16)) * 16
        gb0[r, pl.ds(k, 16)] = zero16
        return 0
    lax.fori_loop(0, CHUNK * (D // 16), _zg, 0)

    def _z1(t, _):
        zb[t, pl.ds(0, 16)] = zero16
        ones_v[t, pl.ds(0, 16)] = one16
        return 0
    lax.fori_loop(0, CHUNK, _z1, 0)

    nbase = sid * NSPLIT
    nz = jnp.where(last, NLAST // CHUNK, NSPLIT // CHUNK)

    def _zacc(t, _):
        pltpu.sync_copy(gb0, acc.at[pl.ds(nbase + t * CHUNK, CHUNK)])
        pltpu.sync_copy(zb, dacc.at[pl.ds(nbase + t * CHUNK, CHUNK)])
        return 0
    lax.fori_loop(0, nz, _zacc, 0)

    @pl.when(last)
    def _():
        tb = nbase + (NLAST // CHUNK) * CHUNK
        pltpu.sync_copy(gb0.at[pl.ds(0, NLAST % CHUNK)], acc.at[pl.ds(tb, NLAST % CHUNK)])
        pltpu.sync_copy(zb.at[pl.ds(0, NLAST % CHUNK)], dacc.at[pl.ds(tb, NLAST % CHUNK)])

    plsc.subcore_barrier()

    nstages = jnp.where(last, ROWS_LAST // STAGE, ROWS_TILE // STAGE)

    def _stage(st, _):
        rb = sid * ROWS_TILE + st * STAGE
        pltpu.sync_copy(row_hbm.at[pl.ds(rb, STAGE)], row_v)

        @pl.when(cid == 0)
        def _():
            pltpu.sync_copy(col_hbm.at[pl.ds(rb, STAGE)], col_v)

        @pl.when(cid == 1)
        def _():
            pltpu.sync_copy(colN_hbm.at[pl.ds(rb, STAGE)], col_v)

        def _ring(kk, _):
            for b in range(RING):
                j = kk * RING + b

                @pl.when(kk > 0)
                def _(b=b, j=j):
                    pltpu.make_async_copy(gbufs[b], acc.at[row_v.at[j]],
                                          ssems.at[b]).wait()
                pltpu.async_copy(xs_hbm.at[col_v.at[j]], gbufs[b], gsems.at[b])
            for b in range(RING):
                j = kk * RING + b
                pltpu.make_async_copy(xs_hbm.at[col_v.at[j]], gbufs[b],
                                      gsems.at[b]).wait()
                pltpu.async_copy(gbufs[b], acc.at[row_v.at[j]], ssems.at[b],
                                 add=True)

                @pl.when(cid == 0)
                def _(j=j):
                    pltpu.async_copy(ones_v, dacc.at[row_v.at[j]], dsem, add=True)
            return 0
        lax.fori_loop(0, STAGE // RING, _ring, 0)

        for b in range(RING):
            pltpu.make_async_copy(gbufs[b], acc.at[row_v.at[STAGE - RING + b]],
                                  ssems.at[b]).wait()
        return 0
    lax.fori_loop(0, nstages, _stage, 0)

    @pl.when(cid == 0)
    def _():
        def _dr(t, _):
            pltpu.make_async_copy(ones_v, dacc.at[pl.ds(0, CHUNK)], dsem).wait()
            return 0
        lax.fori_loop(0, nstages * STAGE, _dr, 0)

    plsc.subcore_barrier()

    @pl.when((cid == 0) & jnp.logical_not(last))
    def _():
        pltpu.sync_copy(acc.at[pl.ds(nbase, NSPLIT)], s1_hbm.at[pl.ds(nbase, NSPLIT)])
        pltpu.sync_copy(dacc.at[pl.ds(nbase, NSPLIT)], deg_hbm.at[pl.ds(nbase, NSPLIT)])

    @pl.when((cid == 0) & last)
    def _():
        pltpu.sync_copy(acc.at[pl.ds(nbase, NLAST)], s1_hbm.at[pl.ds(nbase, NLAST)])
        pltpu.sync_copy(dacc.at[pl.ds(nbase, NLAST)], deg_hbm.at[pl.ds(nbase, NLAST)])

    @pl.when((cid == 1) & jnp.logical_not(last))
    def _():
        pltpu.sync_copy(acc.at[pl.ds(nbase, NSPLIT)], s2_hbm.at[pl.ds(nbase, NSPLIT)])

    @pl.when((cid == 1) & last)
    def _():
        pltpu.sync_copy(acc.at[pl.ds(nbase, NLAST)], s2_hbm.at[pl.ds(nbase, NLAST)])


_seg_kernel = functools.partial(
    pl.kernel,
    out_type=(jax.ShapeDtypeStruct((N, D), jnp.float32),
              jax.ShapeDtypeStruct((N, D), jnp.float32),
              jax.ShapeDtypeStruct((N, 16), jnp.float32)),
    mesh=plsc.VectorSubcoreMesh(core_axis_name="c", subcore_axis_name="s",
                                num_cores=NC, num_subcores=NS),
    compiler_params=pltpu.CompilerParams(use_tc_tiling_on_sc=False),
    scratch_types=[
        pltpu.VMEM((STAGE, CHUNK), jnp.int32),
        pltpu.VMEM((STAGE, CHUNK), jnp.int32),
        pltpu.VMEM((CHUNK, D), jnp.float32),
        pltpu.VMEM((CHUNK, D), jnp.float32),
        pltpu.VMEM((CHUNK, D), jnp.float32),
        pltpu.VMEM((CHUNK, D), jnp.float32),
        pltpu.VMEM((CHUNK, 16), jnp.float32),
        pltpu.VMEM((CHUNK, 16), jnp.float32),
        pltpu.VMEM_SHARED((N, D), jnp.float32),
        pltpu.VMEM_SHARED((N, 16), jnp.float32),
        pltpu.SemaphoreType.DMA((RING,)),
        pltpu.SemaphoreType.DMA((RING,)),
        pltpu.SemaphoreType.DMA,
    ],
)(_seg_body)


def _epi_body(avg_ref, s1_ref, s2_ref, deg_ref, x_ref,
              wid_ref, wamp_ref, watt_ref, bias_ref, out_ref):
    avg = avg_ref[0, 0]
    s1 = s1_ref[...]
    s2 = s2_ref[...]
    x = x_ref[...]
    std = jnp.sqrt(jnp.maximum(s2 - s1 * s1, 0.0) + EPS)
    logd = jnp.log(deg_ref[:, 0:1] + 1.0)
    amp = logd / avg
    att = avg / (logd + EPS)

    f32 = jnp.float32
    h_id = (jnp.dot(s1, wid_ref[0:D], preferred_element_type=f32)
            + jnp.dot(std, wid_ref[D:2 * D], preferred_element_type=f32)
            + jnp.dot(x, wid_ref[2 * D:3 * D], preferred_element_type=f32))
    h_amp = (jnp.dot(s1, wamp_ref[0:D], preferred_element_type=f32)
             + jnp.dot(std, wamp_ref[D:2 * D], preferred_element_type=f32))
    h_att = (jnp.dot(s1, watt_ref[0:D], preferred_element_type=f32)
             + jnp.dot(std, watt_ref[D:2 * D], preferred_element_type=f32))
    out_ref[...] = h_id + amp * h_amp + att * h_att + bias_ref[...]


def _epilogue(avg, s1, s2, deg, x, w_id, w_amp, w_att, bias):
    blk = 2000
    return pl.pallas_call(
        _epi_body,
        grid=(N // blk,),
        in_specs=[
            pl.BlockSpec(memory_space=pltpu.SMEM),
            pl.BlockSpec((blk, D), lambda i: (i, 0)),
            pl.BlockSpec((blk, D), lambda i: (i, 0)),
            pl.BlockSpec((blk, 16), lambda i: (i, 0)),
            pl.BlockSpec((blk, D), lambda i: (i, 0)),
            pl.BlockSpec((3 * D, OUT), lambda i: (0, 0)),
            pl.BlockSpec((2 * D, OUT), lambda i: (0, 0)),
            pl.BlockSpec((2 * D, OUT), lambda i: (0, 0)),
            pl.BlockSpec((1, OUT), lambda i: (0, 0)),
        ],
        out_specs=pl.BlockSpec((blk, OUT), lambda i: (i, 0)),
        out_shape=jax.ShapeDtypeStruct((N, OUT), jnp.float32),
    )(avg, s1, s2, deg, x, w_id, w_amp, w_att, bias)


def kernel(x, edge_index, avg_deg_log, W_pre, W_lin, bias):
    row2 = edge_index[0].reshape(NROWS, CHUNK)
    col2 = edge_index[1].reshape(NROWS, CHUNK)
    colN2 = col2 + N

    xs = _square_stack(x).reshape(2 * N, D)
    s1, s2, deg = _seg_kernel(xs, row2, col2, colN2)

    w_id = jnp.concatenate([W_pre[0].T, W_pre[3].T, W_lin.T], axis=0)
    w_amp = jnp.concatenate([W_pre[1].T, W_pre[4].T], axis=0)
    w_att = jnp.concatenate([W_pre[2].T, W_pre[5].T], axis=0)
    avg = jnp.reshape(avg_deg_log, (1, 1))
    return _epilogue(avg, s1, s2, deg, x,
                     w_id, w_amp, w_att, bias.reshape(1, OUT))
